# baseline (speedup 1.0000x reference)
import functools

import jax
import jax.numpy as jnp
from jax import lax
from jax.experimental import pallas as pl
from jax.experimental.pallas import tpu as pltpu
from jax.experimental.pallas import tpu_sc as plsc

N = 10000
NPAD = 10240
C = 128
NC = 2
NS = 16
NW = NC * NS
ECH = 80
RPS = NPAD // NS
TBN = 1000
ZROW = 4 * N


def _table_body(cf_ref, x_ref, o_ref):
    i = pl.program_id(0)
    d = i // (N // TBN)
    lanes = lax.broadcasted_iota(jnp.int32, (1, 8), 1)
    cv = jnp.sum(jnp.where(lanes == d, cf_ref[...], 0.0))
    o_ref[...] = x_ref[...] * cv


def _scaled_table(x, hop_coef):
    nb = N // TBN
    cf = jnp.zeros((1, 8), jnp.float32).at[0, :4].set(hop_coef)
    return pl.pallas_call(
        _table_body,
        grid=(4 * nb + 1,),
        in_specs=[
            pl.BlockSpec((1, 8), lambda i: (0, 0)),
            pl.BlockSpec((TBN, C), lambda i: (lax.rem(i, nb), 0)),
        ],
        out_specs=pl.BlockSpec((TBN, C), lambda i: (i, 0)),
        out_shape=jax.ShapeDtypeStruct((4 * N + TBN, C), jnp.float32),
    )(cf, x)


def _sc_partials(table, packed, epw):
    nchunk = epw // ECH
    ntrip = (nchunk - 2) // 3
    mesh = plsc.VectorSubcoreMesh(core_axis_name="c", subcore_axis_name="s")

    @functools.partial(
        pl.kernel,
        out_type=jax.ShapeDtypeStruct((2, NPAD, C), jnp.float32),
        mesh=mesh,
        scratch_types=[
            pltpu.VMEM((epw,), jnp.int32),
            [dict(rows=pltpu.VMEM((ECH, C), jnp.float32),
                  src=pltpu.VMEM((ECH,), jnp.int32),
                  dst=pltpu.VMEM((ECH,), jnp.int32),
                  gsem=pltpu.SemaphoreType.DMA,
                  ssem=pltpu.SemaphoreType.DMA) for _ in range(3)],
            pltpu.VMEM_SHARED((NPAD, C), jnp.float32),
        ],
        compiler_params=pltpu.CompilerParams(needs_layout_passes=False),
    )
    def sc_kernel(tb_hbm, pk_hbm, out_hbm, pk_v, bufs, agg_sh):
        cid = lax.axis_index("c")
        sid = lax.axis_index("s")
        wid = sid * NC + cid
        b0, b1, b2 = bufs

        pltpu.sync_copy(pk_hbm.at[wid], pk_v)

        zeros16 = jnp.zeros((16,), jnp.float32)

        def zrow(i, carry):
            for j in range(C // 16):
                b0["rows"][i, pl.ds(j * 16, 16)] = zeros16
            return carry

        lax.fori_loop(0, ECH, zrow, 0)
        row0 = sid * RPS
        for k in range(RPS // ECH):
            pltpu.sync_copy(b0["rows"],
                            agg_sh.at[pl.ds(row0 + k * ECH, ECH)])
        plsc.subcore_barrier()

        def decode(c_i, b):
            cbase = c_i * ECH
            for g in range(ECH // 16):
                pg = pk_v[pl.ds(cbase + g * 16, 16)]
                w = (pg >> 28) & 15
                gidx = (pg & 16383) + w * N - N
                b["src"][pl.ds(g * 16, 16)] = jnp.where(w == 0, ZROW, gidx)
                b["dst"][pl.ds(g * 16, 16)] = (pg >> 14) & 16383

        def gather(b):
            pltpu.async_copy(tb_hbm.at[b["src"]], b["rows"], b["gsem"])

        def gwait(b):
            pltpu.make_async_copy(tb_hbm.at[b["src"]], b["rows"],
                                  b["gsem"]).wait()

        def scatter(b):
            pltpu.async_copy(b["rows"], agg_sh.at[b["dst"]], b["ssem"],
                             add=True)

        def swait(b):
            pltpu.make_async_copy(b["rows"], agg_sh.at[b["dst"]],
                                  b["ssem"]).wait()

        def step(c_i, bx, bz, first=False):
            gwait(bx)
            if not first:
                swait(bz)
            decode(jnp.minimum(c_i + 2, nchunk - 1), bz)
            gather(bz)
            scatter(bx)

        decode(0, b0)
        gather(b0)
        decode(1, b1)
        gather(b1)
        step(jnp.int32(0), b0, b2, first=True)

        def trip(t, carry):
            c_i = 3 * t + 1
            step(c_i, b1, b0)
            step(c_i + 1, b2, b1)
            step(c_i + 2, b0, b2)
            return carry

        lax.fori_loop(0, ntrip, trip, 0)

        gwait(b1)
        swait(b0)
        pltpu.sync_copy(b1["rows"], agg_sh.at[b1["dst"]], add=True)
        gwait(b2)

        plsc.subcore_barrier()
        pltpu.sync_copy(agg_sh.at[pl.ds(row0, RPS)],
                        out_hbm.at[cid].at[pl.ds(row0, RPS)])

    return sc_kernel(table, packed)


def _mlp_body(x_ref, p0_ref, p1_ref, w1_ref, b1_ref, w2_ref, b2_ref, o_ref):
    agg = x_ref[...] + p0_ref[0] + p1_ref[0]
    h = jnp.dot(agg, w1_ref[...], preferred_element_type=jnp.float32)
    h = jnp.maximum(h + b1_ref[...], 0.0)
    o = jnp.dot(h, w2_ref[...], preferred_element_type=jnp.float32)
    o_ref[...] = jnp.maximum(o + b2_ref[...], 0.0)


def _mlp(x, partials, W1, b1, W2, b2):
    BN = 1000
    grid = (N // BN,)
    return pl.pallas_call(
        _mlp_body,
        grid=grid,
        in_specs=[
            pl.BlockSpec((BN, C), lambda i: (i, 0)),
            pl.BlockSpec((1, BN, C), lambda i: (0, i, 0)),
            pl.BlockSpec((1, BN, C), lambda i: (1, i, 0)),
            pl.BlockSpec((C, C), lambda i: (0, 0)),
            pl.BlockSpec((1, C), lambda i: (0, 0)),
            pl.BlockSpec((C, C), lambda i: (0, 0)),
            pl.BlockSpec((1, C), lambda i: (0, 0)),
        ],
        out_specs=pl.BlockSpec((BN, C), lambda i: (i, 0)),
        out_shape=jax.ShapeDtypeStruct((N, C), jnp.float32),
    )(x, partials, partials, W1, b1.reshape(1, C), W2, b2.reshape(1, C))


def kernel(t, node_embeddings, edge_index, edge_weights, W1, b1, W2, b2,
           hop_coef):
    x = jnp.take(node_embeddings, t, axis=0)
    E = edge_weights.shape[0]
    epw = E // NW
    dst = edge_index[0]
    src = edge_index[1]
    packed = (src | (dst << 14) | (edge_weights << 28)).reshape(NW, epw)
    table = _scaled_table(x, hop_coef)
    partials = _sc_partials(table, packed, epw)
    return _mlp(x, partials, W1, b1, W2, b2)


# speedup vs baseline: 14.4441x; 14.4441x over previous
import functools

import jax
import jax.numpy as jnp
from jax import lax
from jax.experimental import pallas as pl
from jax.experimental.pallas import tpu as pltpu
from jax.experimental.pallas import tpu_sc as plsc

N = 10000
NPAD = 10240
C = 128
NC = 2
NS = 16
NW = NC * NS
ECH = 80
RPS = NPAD // NS
TBN = 1000


def _table_body(cf_ref, x_ref, o_ref):
    i = pl.program_id(0)
    d = i // (N // TBN)
    lanes = lax.broadcasted_iota(jnp.int32, (1, 8), 1)
    cv = jnp.sum(jnp.where(lanes == d, cf_ref[...], 0.0))
    o_ref[...] = x_ref[...] * cv


def _scaled_table(x, hop_coef):
    nb = N // TBN
    cf = jnp.zeros((1, 8), jnp.float32).at[0, :4].set(hop_coef)
    return pl.pallas_call(
        _table_body,
        grid=(4 * nb,),
        in_specs=[
            pl.BlockSpec((1, 8), lambda i: (0, 0)),
            pl.BlockSpec((TBN, C), lambda i: (lax.rem(i, nb), 0)),
        ],
        out_specs=pl.BlockSpec((TBN, C), lambda i: (i, 0)),
        out_shape=jax.ShapeDtypeStruct((4 * N, C), jnp.float32),
    )(cf, x)


def _sc_partials(table, packed, epw):
    nchunk = epw // ECH
    ntrip = (nchunk - 2) // 3
    mesh = plsc.VectorSubcoreMesh(core_axis_name="c", subcore_axis_name="s")

    @functools.partial(
        pl.kernel,
        out_type=jax.ShapeDtypeStruct((2, NPAD, C), jnp.float32),
        mesh=mesh,
        scratch_types=[
            pltpu.VMEM((epw,), jnp.int32),
            [dict(rows=pltpu.VMEM((ECH, C), jnp.float32),
                  src=pltpu.VMEM((ECH,), jnp.int32),
                  dst=pltpu.VMEM((ECH,), jnp.int32),
                  gsem=pltpu.SemaphoreType.DMA,
                  ssem=pltpu.SemaphoreType.DMA) for _ in range(3)],
            pltpu.VMEM_SHARED((NPAD, C), jnp.float32),
        ],
        compiler_params=pltpu.CompilerParams(needs_layout_passes=False),
    )
    def sc_kernel(tb_hbm, pk_hbm, out_hbm, pk_v, bufs, agg_sh):
        cid = lax.axis_index("c")
        sid = lax.axis_index("s")
        wid = sid * NC + cid
        b0, b1, b2 = bufs

        pltpu.sync_copy(pk_hbm.at[wid], pk_v)

        zeros16 = jnp.zeros((16,), jnp.float32)

        def zrow(i, carry):
            for j in range(C // 16):
                b0["rows"][i, pl.ds(j * 16, 16)] = zeros16
            return carry

        lax.fori_loop(0, ECH, zrow, 0)
        row0 = sid * RPS
        for k in range(RPS // ECH):
            pltpu.sync_copy(b0["rows"],
                            agg_sh.at[pl.ds(row0 + k * ECH, ECH)])
        plsc.subcore_barrier()

        def decode(c_i, b):
            cbase = c_i * ECH
            for g in range(ECH // 16):
                pg = pk_v[pl.ds(cbase + g * 16, 16)]
                w = (pg >> 28) & 15
                srcv = pg & 16383
                gidx = srcv + w * N - N
                dstv = (pg >> 14) & 16383
                b["src"][pl.ds(g * 16, 16)] = jnp.where(w == 0, srcv, gidx)
                b["dst"][pl.ds(g * 16, 16)] = jnp.where(
                    w == 0, N + (srcv & 127), dstv)

        def gather(b):
            pltpu.async_copy(tb_hbm.at[b["src"]], b["rows"], b["gsem"])

        def gwait(b):
            pltpu.make_async_copy(tb_hbm.at[b["src"]], b["rows"],
                                  b["gsem"]).wait()

        def scatter(b):
            pltpu.async_copy(b["rows"], agg_sh.at[b["dst"]], b["ssem"],
                             add=True)

        def swait(b):
            pltpu.make_async_copy(b["rows"], agg_sh.at[b["dst"]],
                                  b["ssem"]).wait()

        def step(c_i, bx, bz, first=False):
            gwait(bx)
            if not first:
                swait(bz)
            decode(jnp.minimum(c_i + 2, nchunk - 1), bz)
            gather(bz)
            scatter(bx)

        decode(0, b0)
        gather(b0)
        decode(1, b1)
        gather(b1)
        step(jnp.int32(0), b0, b2, first=True)

        def trip(t, carry):
            c_i = 3 * t + 1
            step(c_i, b1, b0)
            step(c_i + 1, b2, b1)
            step(c_i + 2, b0, b2)
            return carry

        lax.fori_loop(0, ntrip, trip, 0)

        gwait(b1)
        swait(b0)
        pltpu.sync_copy(b1["rows"], agg_sh.at[b1["dst"]], add=True)
        gwait(b2)

        plsc.subcore_barrier()
        pltpu.sync_copy(agg_sh.at[pl.ds(row0, RPS)],
                        out_hbm.at[cid].at[pl.ds(row0, RPS)])

    return sc_kernel(table, packed)


def _mlp_body(x_ref, p0_ref, p1_ref, w1_ref, b1_ref, w2_ref, b2_ref, o_ref):
    agg = x_ref[...] + p0_ref[0] + p1_ref[0]
    h = jnp.dot(agg, w1_ref[...], preferred_element_type=jnp.float32)
    h = jnp.maximum(h + b1_ref[...], 0.0)
    o = jnp.dot(h, w2_ref[...], preferred_element_type=jnp.float32)
    o_ref[...] = jnp.maximum(o + b2_ref[...], 0.0)


def _mlp(x, partials, W1, b1, W2, b2):
    BN = 1000
    grid = (N // BN,)
    return pl.pallas_call(
        _mlp_body,
        grid=grid,
        in_specs=[
            pl.BlockSpec((BN, C), lambda i: (i, 0)),
            pl.BlockSpec((1, BN, C), lambda i: (0, i, 0)),
            pl.BlockSpec((1, BN, C), lambda i: (1, i, 0)),
            pl.BlockSpec((C, C), lambda i: (0, 0)),
            pl.BlockSpec((1, C), lambda i: (0, 0)),
            pl.BlockSpec((C, C), lambda i: (0, 0)),
            pl.BlockSpec((1, C), lambda i: (0, 0)),
        ],
        out_specs=pl.BlockSpec((BN, C), lambda i: (i, 0)),
        out_shape=jax.ShapeDtypeStruct((N, C), jnp.float32),
    )(x, partials, partials, W1, b1.reshape(1, C), W2, b2.reshape(1, C))


def kernel(t, node_embeddings, edge_index, edge_weights, W1, b1, W2, b2,
           hop_coef):
    x = jnp.take(node_embeddings, t, axis=0)
    E = edge_weights.shape[0]
    epw = E // NW
    dst = edge_index[0]
    src = edge_index[1]
    packed = (src | (dst << 14) | (edge_weights << 28)).reshape(NW, epw)
    table = _scaled_table(x, hop_coef)
    partials = _sc_partials(table, packed, epw)
    return _mlp(x, partials, W1, b1, W2, b2)


# speedup vs baseline: 15.0737x; 1.0436x over previous
import functools

import jax
import jax.numpy as jnp
from jax import lax
from jax.experimental import pallas as pl
from jax.experimental.pallas import tpu as pltpu
from jax.experimental.pallas import tpu_sc as plsc

N = 10000
NPAD = 10240
C = 128
NC = 2
NS = 16
NW = NC * NS
ECH = 80
RPS = NPAD // NS
TBN = 1000


def _table_body(cf_ref, x_ref, o_ref):
    i = pl.program_id(0)
    d = i // (N // TBN)
    lanes = lax.broadcasted_iota(jnp.int32, (1, 8), 1)
    cv = jnp.sum(jnp.where(lanes == d, cf_ref[...], 0.0))
    o_ref[...] = x_ref[...] * cv


def _scaled_table(x, hop_coef):
    nb = N // TBN
    cf = jnp.zeros((1, 8), jnp.float32).at[0, :4].set(hop_coef)
    return pl.pallas_call(
        _table_body,
        grid=(4 * nb,),
        in_specs=[
            pl.BlockSpec((1, 8), lambda i: (0, 0)),
            pl.BlockSpec((TBN, C), lambda i: (lax.rem(i, nb), 0)),
        ],
        out_specs=pl.BlockSpec((TBN, C), lambda i: (i, 0)),
        out_shape=jax.ShapeDtypeStruct((4 * N, C), jnp.float32),
    )(cf, x)


def _sc_partials(table, packed, epw):
    nchunk = epw // ECH
    ntrip = (nchunk - 2) // 3
    mesh = plsc.VectorSubcoreMesh(core_axis_name="c", subcore_axis_name="s")

    @functools.partial(
        pl.kernel,
        out_type=jax.ShapeDtypeStruct((2, NPAD, C), jnp.float32),
        mesh=mesh,
        scratch_types=[
            pltpu.VMEM((epw + 400,), jnp.int32),
            [dict(rows=pltpu.VMEM((ECH, C), jnp.float32),
                  src=pltpu.VMEM((ECH,), jnp.int32),
                  dst=pltpu.VMEM((ECH,), jnp.int32),
                  gsem=pltpu.SemaphoreType.DMA,
                  ssem=pltpu.SemaphoreType.DMA) for _ in range(3)],
            pltpu.VMEM_SHARED((NPAD, C), jnp.float32),
        ],
        compiler_params=pltpu.CompilerParams(needs_layout_passes=False),
    )
    def sc_kernel(tb_hbm, pk_hbm, out_hbm, pk_v, bufs, agg_sh):
        cid = lax.axis_index("c")
        sid = lax.axis_index("s")
        wid = sid * NC + cid
        b0, b1, b2 = bufs

        pltpu.sync_copy(pk_hbm.at[wid], pk_v)

        zeros16 = jnp.zeros((16,), jnp.float32)

        def zrow(i, carry):
            for j in range(C // 16):
                b0["rows"][i, pl.ds(j * 16, 16)] = zeros16
            return carry

        lax.fori_loop(0, ECH, zrow, 0)
        row0 = sid * RPS
        for k in range(RPS // ECH):
            pltpu.sync_copy(b0["rows"],
                            agg_sh.at[pl.ds(row0 + k * ECH, ECH)])
        plsc.subcore_barrier()

        def cgroup(g, cnt):
            pg = pk_v[pl.ds(g * 16, 16)]
            keep = ((pg >> 28) & 15) > 0
            plsc.store_compressed(pk_v.at[pl.ds(cnt, 16)], pg, mask=keep)
            npop = plsc.all_reduce_population_count(keep)
            return cnt + jnp.max(npop)

        cnt = lax.fori_loop(0, epw // 16, cgroup, jnp.int32(0))

        ones = jnp.full((16,), True)
        for k in range(400 // 16):
            pad = lax.iota(jnp.int32, 16) + (16 * k)
            plsc.store_compressed(pk_v.at[pl.ds(cnt + 16 * k, 16)], pad, mask=ones)

        cc = (cnt + (ECH - 1)) // ECH
        ntrip_d = jnp.maximum(cc // 3, 1)

        def decode(c_i, b):
            cbase = c_i * ECH
            for g in range(ECH // 16):
                pg = pk_v[pl.ds(cbase + g * 16, 16)]
                w = (pg >> 28) & 15
                srcv = pg & 16383
                gidx = srcv + w * N - N
                dstv = (pg >> 14) & 16383
                b["src"][pl.ds(g * 16, 16)] = jnp.where(w == 0, srcv, gidx)
                b["dst"][pl.ds(g * 16, 16)] = jnp.where(
                    w == 0, N + (srcv & 127), dstv)

        def gather(b):
            pltpu.async_copy(tb_hbm.at[b["src"]], b["rows"], b["gsem"])

        def gwait(b):
            pltpu.make_async_copy(tb_hbm.at[b["src"]], b["rows"],
                                  b["gsem"]).wait()

        def scatter(b):
            pltpu.async_copy(b["rows"], agg_sh.at[b["dst"]], b["ssem"],
                             add=True)

        def swait(b):
            pltpu.make_async_copy(b["rows"], agg_sh.at[b["dst"]],
                                  b["ssem"]).wait()

        last = 3 * ntrip_d + 1

        def step(c_i, bx, bz, first=False):
            gwait(bx)
            if not first:
                swait(bz)
            decode(jnp.minimum(c_i + 2, last), bz)
            gather(bz)
            scatter(bx)

        decode(0, b0)
        gather(b0)
        decode(1, b1)
        gather(b1)
        step(jnp.int32(0), b0, b2, first=True)

        def trip(t, carry):
            c_i = 3 * t + 1
            step(c_i, b1, b0)
            step(c_i + 1, b2, b1)
            step(c_i + 2, b0, b2)
            return carry

        lax.fori_loop(0, ntrip_d, trip, 0)

        gwait(b1)
        swait(b0)
        pltpu.sync_copy(b1["rows"], agg_sh.at[b1["dst"]], add=True)
        gwait(b2)

        plsc.subcore_barrier()
        pltpu.sync_copy(agg_sh.at[pl.ds(row0, RPS)],
                        out_hbm.at[cid].at[pl.ds(row0, RPS)])

    return sc_kernel(table, packed)


def _mlp_body(x_ref, p0_ref, p1_ref, w1_ref, b1_ref, w2_ref, b2_ref, o_ref):
    agg = x_ref[...] + p0_ref[0] + p1_ref[0]
    h = jnp.dot(agg, w1_ref[...], preferred_element_type=jnp.float32)
    h = jnp.maximum(h + b1_ref[...], 0.0)
    o = jnp.dot(h, w2_ref[...], preferred_element_type=jnp.float32)
    o_ref[...] = jnp.maximum(o + b2_ref[...], 0.0)


def _mlp(x, partials, W1, b1, W2, b2):
    BN = 1000
    grid = (N // BN,)
    return pl.pallas_call(
        _mlp_body,
        grid=grid,
        in_specs=[
            pl.BlockSpec((BN, C), lambda i: (i, 0)),
            pl.BlockSpec((1, BN, C), lambda i: (0, i, 0)),
            pl.BlockSpec((1, BN, C), lambda i: (1, i, 0)),
            pl.BlockSpec((C, C), lambda i: (0, 0)),
            pl.BlockSpec((1, C), lambda i: (0, 0)),
            pl.BlockSpec((C, C), lambda i: (0, 0)),
            pl.BlockSpec((1, C), lambda i: (0, 0)),
        ],
        out_specs=pl.BlockSpec((BN, C), lambda i: (i, 0)),
        out_shape=jax.ShapeDtypeStruct((N, C), jnp.float32),
    )(x, partials, partials, W1, b1.reshape(1, C), W2, b2.reshape(1, C))


def kernel(t, node_embeddings, edge_index, edge_weights, W1, b1, W2, b2,
           hop_coef):
    x = jnp.take(node_embeddings, t, axis=0)
    E = edge_weights.shape[0]
    epw = E // NW
    dst = edge_index[0]
    src = edge_index[1]
    packed = (src | (dst << 14) | (edge_weights << 28)).reshape(NW, epw)
    packed = jnp.pad(packed, ((0, 0), (0, 400)))
    table = _scaled_table(x, hop_coef)
    partials = _sc_partials(table, packed, epw)
    return _mlp(x, partials, W1, b1, W2, b2)


# speedup vs baseline: 15.4511x; 1.0250x over previous
import functools

import jax
import jax.numpy as jnp
from jax import lax
from jax.experimental import pallas as pl
from jax.experimental.pallas import tpu as pltpu
from jax.experimental.pallas import tpu_sc as plsc

N = 10000
NPAD = 10240
C = 128
NC = 2
NS = 16
NW = NC * NS
ECH = 80
RPS = NPAD // NS
TBN = 1000


def _sc_partials(x, packed, lut, epw):
    nchunk = epw // ECH
    ntrip = (nchunk - 2) // 3
    mesh = plsc.VectorSubcoreMesh(core_axis_name="c", subcore_axis_name="s")

    @functools.partial(
        pl.kernel,
        out_type=jax.ShapeDtypeStruct((2, NPAD, C), jnp.float32),
        mesh=mesh,
        scratch_types=[
            pltpu.VMEM((epw + 400,), jnp.int32),
            pltpu.VMEM((16,), jnp.float32),
            [dict(rows=pltpu.VMEM((ECH, C), jnp.float32),
                  src=pltpu.VMEM((ECH,), jnp.int32),
                  dst=pltpu.VMEM((ECH,), jnp.int32),
                  cf=pltpu.VMEM((ECH,), jnp.float32),
                  gsem=pltpu.SemaphoreType.DMA,
                  ssem=pltpu.SemaphoreType.DMA) for _ in range(3)],
            pltpu.VMEM_SHARED((NPAD, C), jnp.float32),
        ],
        compiler_params=pltpu.CompilerParams(needs_layout_passes=False),
    )
    def sc_kernel(x_hbm, pk_hbm, lut_hbm, out_hbm, pk_v, lut_v, bufs, agg_sh):
        cid = lax.axis_index("c")
        sid = lax.axis_index("s")
        wid = sid * NC + cid
        b0, b1, b2 = bufs

        pltpu.sync_copy(lut_hbm, lut_v)
        pltpu.sync_copy(pk_hbm.at[wid], pk_v)

        zeros16 = jnp.zeros((16,), jnp.float32)

        def zrow(i, carry):
            for j in range(C // 16):
                b0["rows"][i, pl.ds(j * 16, 16)] = zeros16
            return carry

        lax.fori_loop(0, ECH, zrow, 0)
        row0 = sid * RPS
        for k in range(RPS // ECH):
            pltpu.sync_copy(b0["rows"],
                            agg_sh.at[pl.ds(row0 + k * ECH, ECH)])
        plsc.subcore_barrier()

        def cgroup(g, cnt):
            pg = pk_v[pl.ds(g * 16, 16)]
            keep = ((pg >> 28) & 15) > 0
            plsc.store_compressed(pk_v.at[pl.ds(cnt, 16)], pg, mask=keep)
            npop = plsc.all_reduce_population_count(keep)
            return cnt + jnp.max(npop)

        cnt = lax.fori_loop(0, epw // 16, cgroup, jnp.int32(0))

        ones = jnp.full((16,), True)
        for k in range(400 // 16):
            pad = lax.iota(jnp.int32, 16) + (16 * k)
            plsc.store_compressed(pk_v.at[pl.ds(cnt + 16 * k, 16)], pad, mask=ones)

        cc = (cnt + (ECH - 1)) // ECH
        ntrip_d = jnp.maximum(cc // 3, 1)

        def decode(c_i, b):
            cbase = c_i * ECH
            for g in range(ECH // 16):
                pg = pk_v[pl.ds(cbase + g * 16, 16)]
                w = (pg >> 28) & 15
                srcv = pg & 16383
                dstv = (pg >> 14) & 16383
                b["src"][pl.ds(g * 16, 16)] = srcv
                b["dst"][pl.ds(g * 16, 16)] = jnp.where(
                    w == 0, N + (srcv & 127), dstv)
                b["cf"][pl.ds(g * 16, 16)] = plsc.load_gather(lut_v, [w])

        def scale(b):
            rows, cf = b["rows"], b["cf"]

            def srow(q, rcarry):
                r = 2 * q
                cv0 = plsc.load_gather(cf, [jnp.full((16,), 0, jnp.int32)
                                            + r])
                cv1 = plsc.load_gather(cf, [jnp.full((16,), 0, jnp.int32)
                                            + (r + 1)])
                for j in range(C // 16):
                    rows[r, pl.ds(j * 16, 16)] = (
                        rows[r, pl.ds(j * 16, 16)] * cv0)
                for j in range(C // 16):
                    rows[r + 1, pl.ds(j * 16, 16)] = (
                        rows[r + 1, pl.ds(j * 16, 16)] * cv1)
                return rcarry

            lax.fori_loop(0, ECH // 2, srow, 0)

        def gather(b):
            pltpu.async_copy(x_hbm.at[b["src"]], b["rows"], b["gsem"])

        def gwait(b):
            pltpu.make_async_copy(x_hbm.at[b["src"]], b["rows"],
                                  b["gsem"]).wait()

        def scatter(b):
            pltpu.async_copy(b["rows"], agg_sh.at[b["dst"]], b["ssem"],
                             add=True)

        def swait(b):
            pltpu.make_async_copy(b["rows"], agg_sh.at[b["dst"]],
                                  b["ssem"]).wait()

        last = 3 * ntrip_d + 1

        def step(c_i, bx, bz, first=False):
            gwait(bx)
            scale(bx)
            if not first:
                swait(bz)
            decode(jnp.minimum(c_i + 2, last), bz)
            gather(bz)
            scatter(bx)

        decode(0, b0)
        gather(b0)
        decode(1, b1)
        gather(b1)
        step(jnp.int32(0), b0, b2, first=True)

        def trip(t, carry):
            c_i = 3 * t + 1
            step(c_i, b1, b0)
            step(c_i + 1, b2, b1)
            step(c_i + 2, b0, b2)
            return carry

        lax.fori_loop(0, ntrip_d, trip, 0)

        gwait(b1)
        scale(b1)
        swait(b0)
        pltpu.sync_copy(b1["rows"], agg_sh.at[b1["dst"]], add=True)
        gwait(b2)

        plsc.subcore_barrier()
        pltpu.sync_copy(agg_sh.at[pl.ds(row0, RPS)],
                        out_hbm.at[cid].at[pl.ds(row0, RPS)])

    return sc_kernel(x, packed, lut)


def _mlp_body(x_ref, p0_ref, p1_ref, w1_ref, b1_ref, w2_ref, b2_ref, o_ref):
    agg = x_ref[...] + p0_ref[0] + p1_ref[0]
    h = jnp.dot(agg, w1_ref[...], preferred_element_type=jnp.float32)
    h = jnp.maximum(h + b1_ref[...], 0.0)
    o = jnp.dot(h, w2_ref[...], preferred_element_type=jnp.float32)
    o_ref[...] = jnp.maximum(o + b2_ref[...], 0.0)


def _mlp(x, partials, W1, b1, W2, b2):
    BN = 1000
    grid = (N // BN,)
    return pl.pallas_call(
        _mlp_body,
        grid=grid,
        in_specs=[
            pl.BlockSpec((BN, C), lambda i: (i, 0)),
            pl.BlockSpec((1, BN, C), lambda i: (0, i, 0)),
            pl.BlockSpec((1, BN, C), lambda i: (1, i, 0)),
            pl.BlockSpec((C, C), lambda i: (0, 0)),
            pl.BlockSpec((1, C), lambda i: (0, 0)),
            pl.BlockSpec((C, C), lambda i: (0, 0)),
            pl.BlockSpec((1, C), lambda i: (0, 0)),
        ],
        out_specs=pl.BlockSpec((BN, C), lambda i: (i, 0)),
        out_shape=jax.ShapeDtypeStruct((N, C), jnp.float32),
    )(x, partials, partials, W1, b1.reshape(1, C), W2, b2.reshape(1, C))


def kernel(t, node_embeddings, edge_index, edge_weights, W1, b1, W2, b2,
           hop_coef):
    x = jnp.take(node_embeddings, t, axis=0)
    E = edge_weights.shape[0]
    epw = E // NW
    dst = edge_index[0]
    src = edge_index[1]
    packed = (src | (dst << 14) | (edge_weights << 28)).reshape(NW, epw)
    packed = jnp.pad(packed, ((0, 0), (0, 400)))
    lut = jnp.zeros((16,), jnp.float32).at[1:5].set(hop_coef)
    partials = _sc_partials(x, packed, lut, epw)
    return _mlp(x, partials, W1, b1, W2, b2)


# speedup vs baseline: 15.7751x; 1.0210x over previous
import functools

import jax
import jax.numpy as jnp
from jax import lax
from jax.experimental import pallas as pl
from jax.experimental.pallas import tpu as pltpu
from jax.experimental.pallas import tpu_sc as plsc

N = 10000
NPAD = 10240
C = 128
NC = 2
NS = 16
NW = NC * NS
ECH = 80
RPS = NPAD // NS
TBN = 1000


def _sc_partials(x, packed, lut, epw):
    nchunk = epw // ECH
    ntrip = (nchunk - 2) // 3
    mesh = plsc.VectorSubcoreMesh(core_axis_name="c", subcore_axis_name="s")

    @functools.partial(
        pl.kernel,
        out_type=jax.ShapeDtypeStruct((2, NPAD, C), jnp.float32),
        mesh=mesh,
        scratch_types=[
            pltpu.VMEM((epw + 400,), jnp.int32),
            pltpu.VMEM((16,), jnp.float32),
            [dict(rows=pltpu.VMEM((ECH, C), jnp.float32),
                  src=pltpu.VMEM((ECH,), jnp.int32),
                  dst=pltpu.VMEM((ECH,), jnp.int32),
                  cf=pltpu.VMEM((ECH,), jnp.float32),
                  gsem=pltpu.SemaphoreType.DMA,
                  ssem=pltpu.SemaphoreType.DMA) for _ in range(3)],
            pltpu.VMEM_SHARED((NPAD, C), jnp.float32),
        ],
        compiler_params=pltpu.CompilerParams(needs_layout_passes=False),
    )
    def sc_kernel(x_hbm, pk_hbm, lut_hbm, out_hbm, pk_v, lut_v, bufs, agg_sh):
        cid = lax.axis_index("c")
        sid = lax.axis_index("s")
        wid = sid * NC + cid
        b0, b1, b2 = bufs

        pltpu.sync_copy(lut_hbm, lut_v)
        pltpu.sync_copy(pk_hbm.at[wid], pk_v)

        zeros16 = jnp.zeros((16,), jnp.float32)

        def zrow(i, carry):
            for j in range(C // 16):
                b0["rows"][i, pl.ds(j * 16, 16)] = zeros16
            return carry

        lax.fori_loop(0, ECH, zrow, 0)
        row0 = sid * RPS
        for k in range(RPS // ECH):
            pltpu.sync_copy(b0["rows"],
                            agg_sh.at[pl.ds(row0 + k * ECH, ECH)])
        plsc.subcore_barrier()

        def cgroup(g, cnt):
            pg = pk_v[pl.ds(g * 16, 16)]
            keep = ((pg >> 28) & 15) > 0
            plsc.store_compressed(pk_v.at[pl.ds(cnt, 16)], pg, mask=keep)
            npop = plsc.all_reduce_population_count(keep)
            return cnt + jnp.max(npop)

        cnt = lax.fori_loop(0, epw // 16, cgroup, jnp.int32(0))

        ones = jnp.full((16,), True)
        for k in range(400 // 16):
            pad = lax.iota(jnp.int32, 16) + (16 * k)
            plsc.store_compressed(pk_v.at[pl.ds(cnt + 16 * k, 16)], pad, mask=ones)

        cc = (cnt + (ECH - 1)) // ECH
        ntrip_d = jnp.maximum(cc // 3, 1)

        def decode(c_i, b):
            cbase = c_i * ECH
            for g in range(ECH // 16):
                pg = pk_v[pl.ds(cbase + g * 16, 16)]
                w = (pg >> 28) & 15
                srcv = pg & 16383
                dstv = (pg >> 14) & 16383
                b["src"][pl.ds(g * 16, 16)] = srcv
                b["dst"][pl.ds(g * 16, 16)] = jnp.where(
                    w == 0, N + (srcv & 127), dstv)
                b["cf"][pl.ds(g * 16, 16)] = plsc.load_gather(lut_v, [w])

        def scale(b):
            rows, cf = b["rows"], b["cf"]
            lanes = lax.iota(jnp.int32, 16)

            def srow(q, rcarry):
                r0 = 4 * q
                cfg = cf[pl.ds((q >> 2) << 4, 16)]
                lane0 = (q & 3) * 4
                for u in range(4):
                    r = r0 + u
                    cv = lax.gather(
                        cfg, (lanes * 0 + (lane0 + u))[:, None],
                        lax.GatherDimensionNumbers(
                            offset_dims=(), collapsed_slice_dims=(0,),
                            start_index_map=(0,)),
                        (1,), mode=lax.GatherScatterMode.PROMISE_IN_BOUNDS)
                    for j in range(C // 16):
                        rows[r, pl.ds(j * 16, 16)] = (
                            rows[r, pl.ds(j * 16, 16)] * cv)
                return rcarry

            lax.fori_loop(0, ECH // 4, srow, 0)

        def gather(b):
            pltpu.async_copy(x_hbm.at[b["src"]], b["rows"], b["gsem"])

        def gwait(b):
            pltpu.make_async_copy(x_hbm.at[b["src"]], b["rows"],
                                  b["gsem"]).wait()

        def scatter(b):
            pltpu.async_copy(b["rows"], agg_sh.at[b["dst"]], b["ssem"],
                             add=True)

        def swait(b):
            pltpu.make_async_copy(b["rows"], agg_sh.at[b["dst"]],
                                  b["ssem"]).wait()

        last = 3 * ntrip_d + 1

        def step(c_i, bx, bz, first=False):
            gwait(bx)
            scale(bx)
            scatter(bx)
            if not first:
                swait(bz)
            decode(jnp.minimum(c_i + 2, last), bz)
            gather(bz)

        decode(0, b0)
        gather(b0)
        decode(1, b1)
        gather(b1)
        step(jnp.int32(0), b0, b2, first=True)

        def trip(t, carry):
            c_i = 3 * t + 1
            step(c_i, b1, b0)
            step(c_i + 1, b2, b1)
            step(c_i + 2, b0, b2)
            return carry

        lax.fori_loop(0, ntrip_d, trip, 0)

        gwait(b1)
        scale(b1)
        swait(b0)
        pltpu.sync_copy(b1["rows"], agg_sh.at[b1["dst"]], add=True)
        gwait(b2)

        plsc.subcore_barrier()
        pltpu.sync_copy(agg_sh.at[pl.ds(row0, RPS)],
                        out_hbm.at[cid].at[pl.ds(row0, RPS)])

    return sc_kernel(x, packed, lut)


def _mlp_body(x_ref, p0_ref, p1_ref, w1_ref, b1_ref, w2_ref, b2_ref, o_ref):
    agg = x_ref[...] + p0_ref[0] + p1_ref[0]
    h = jnp.dot(agg, w1_ref[...], preferred_element_type=jnp.float32)
    h = jnp.maximum(h + b1_ref[...], 0.0)
    o = jnp.dot(h, w2_ref[...], preferred_element_type=jnp.float32)
    o_ref[...] = jnp.maximum(o + b2_ref[...], 0.0)


def _mlp(x, partials, W1, b1, W2, b2):
    BN = 1000
    grid = (N // BN,)
    return pl.pallas_call(
        _mlp_body,
        grid=grid,
        in_specs=[
            pl.BlockSpec((BN, C), lambda i: (i, 0)),
            pl.BlockSpec((1, BN, C), lambda i: (0, i, 0)),
            pl.BlockSpec((1, BN, C), lambda i: (1, i, 0)),
            pl.BlockSpec((C, C), lambda i: (0, 0)),
            pl.BlockSpec((1, C), lambda i: (0, 0)),
            pl.BlockSpec((C, C), lambda i: (0, 0)),
            pl.BlockSpec((1, C), lambda i: (0, 0)),
        ],
        out_specs=pl.BlockSpec((BN, C), lambda i: (i, 0)),
        out_shape=jax.ShapeDtypeStruct((N, C), jnp.float32),
    )(x, partials, partials, W1, b1.reshape(1, C), W2, b2.reshape(1, C))


def kernel(t, node_embeddings, edge_index, edge_weights, W1, b1, W2, b2,
           hop_coef):
    x = jnp.take(node_embeddings, t, axis=0)
    E = edge_weights.shape[0]
    epw = E // NW
    dst = edge_index[0]
    src = edge_index[1]
    packed = (src | (dst << 14) | (edge_weights << 28)).reshape(NW, epw)
    packed = jnp.pad(packed, ((0, 0), (0, 400)))
    lut = jnp.zeros((16,), jnp.float32).at[1:5].set(hop_coef)
    partials = _sc_partials(x, packed, lut, epw)
    return _mlp(x, partials, W1, b1, W2, b2)
